# baseline (speedup 1.0000x reference)
import jax
import jax.numpy as jnp
from jax import lax
from jax.experimental import pallas as pl
from jax.experimental.pallas import tpu as pltpu
from jax.experimental.pallas import tpu_sc as plsc

_N = 512
_B = 16
_ES = 128
_WD = 300
_V = 30000
_RATIO = 0.2
_HI = lax.Precision.HIGHEST
_VCHUNK = 10240

def _proj_body(embt_ref, w1_ref, out_ref):
    out_ref[...] = lax.dot_general(
        embt_ref[...], w1_ref[...],
        dimension_numbers=(((0,), (0,)), ((), ())))


_PROJ_CALL = pl.pallas_call(
    _proj_body,
    grid=(_V // _VCHUNK + (_V % _VCHUNK != 0),),
    in_specs=[
        pl.BlockSpec((_WD, _VCHUNK), lambda j: (0, j)),
        pl.BlockSpec((_WD, _ES), lambda j: (0, 0)),
    ],
    out_specs=pl.BlockSpec((_VCHUNK, _ES), lambda j: (j, 0)),
    out_shape=jax.ShapeDtypeStruct((_V, _ES), jnp.float32),
)

_NW = 32
_RW = _N // _NW


def _sc_gather_body(table, idx_hbm, out, idx_v, rows_v, sem):
    wid = lax.axis_index("s") * 2 + lax.axis_index("c")
    base = wid * _RW
    pltpu.sync_copy(idx_hbm.at[pl.ds(base, _RW)], idx_v)
    pltpu.async_copy(table.at[idx_v], rows_v, sem).wait()
    pltpu.sync_copy(rows_v, out.at[pl.ds(base, _RW)])


def _sc_gather(table, idx):
    k = pl.kernel(
        _sc_gather_body,
        out_type=jax.ShapeDtypeStruct((_N, _ES), jnp.float32),
        mesh=plsc.VectorSubcoreMesh(core_axis_name="c", subcore_axis_name="s"),
        scratch_types=[
            pltpu.VMEM((_RW,), jnp.int32),
            pltpu.VMEM((_RW, _ES), jnp.float32),
            pltpu.SemaphoreType.DMA,
        ],
    )
    return k(table, idx)


def _dense_body(adj_ref, h1_ref, brow_ref,
                b1_ref, w2_ref, b2_ref, wl_ref, bl_ref,
                pwr_ref, pwl_ref, pb_ref,
                f1_ref, fb1_ref, f2_ref, fb2_ref, fl_ref, fbl_ref,
                lfw_ref, lfb_ref, out_ref):
    f32 = jnp.float32
    adj = adj_ref[...]
    w = ((adj + adj.T) > 0).astype(f32)
    deg = jnp.sum(w, axis=1, keepdims=True) + 1.0
    dinv = lax.rsqrt(deg)
    dinv2 = dinv * dinv

    def gcn_h(h, b, dv, dv2, keep):
        v = h * dv
        if keep is not None:
            v = v * keep
        u = jnp.dot(w, v, precision=_HI)
        if keep is not None:
            u = u * keep
        return u * dv + h * dv2 + b

    relu = lambda t: jnp.maximum(t, 0.0)

    x1 = relu(gcn_h(h1_ref[...], b1_ref[...], dinv, dinv2, None))
    h2 = jnp.dot(x1, w2_ref[...])
    x2 = relu(gcn_h(h2, b2_ref[...], dinv, dinv2, None))
    x = relu(jnp.dot(jnp.concatenate([x1, x2], axis=1), wl_ref[...])
             + bl_ref[...])

    brow = brow_ref[...]
    bcol = brow.T
    m_bn = (lax.broadcasted_iota(jnp.int32, (_B, _N), 0) == brow).astype(f32)
    m_nb = (lax.broadcasted_iota(jnp.int32, (_N, _B), 1) == bcol).astype(f32)
    counts = jnp.sum(m_bn, axis=1, keepdims=True)
    xs0 = jnp.dot(m_bn, x, precision=_HI) / jnp.maximum(counts, 1.0)

    aggr = jnp.dot(w, x, precision=_HI)
    score = (jnp.dot(aggr, pwl_ref[...])
             + jnp.dot(x, pwr_ref[...]) + pb_ref[...])
    score_row = score.T

    kk = jnp.ceil(_RATIO * counts)
    k_node = jnp.dot(m_nb, kk, precision=_HI)

    same = bcol == brow
    ii = lax.broadcasted_iota(jnp.int32, (_N, _N), 0)
    jj = lax.broadcasted_iota(jnp.int32, (_N, _N), 1)
    beats = (score_row > score) | ((score_row == score) & (jj < ii))
    rank = jnp.sum(jnp.where(same & beats, 1.0, 0.0), axis=1, keepdims=True)
    keep = (rank < k_node).astype(f32)

    xg = x * jnp.tanh(score)
    deg2 = 1.0 + keep * jnp.dot(w, keep, precision=_HI)
    db = lax.rsqrt(deg2)
    db2 = db * db

    g1 = jnp.dot(xg, f1_ref[...])
    y1 = relu(gcn_h(g1, fb1_ref[...], db, db2, keep))
    g2 = jnp.dot(y1, f2_ref[...])
    y2 = relu(gcn_h(g2, fb2_ref[...], db, db2, keep))
    out2 = jnp.dot(jnp.concatenate([y1, y2], axis=1), fl_ref[...]) + fbl_ref[...]

    c1 = jnp.dot(m_bn, keep, precision=_HI)
    xs1 = jnp.dot(m_bn, out2 * keep, precision=_HI) / jnp.maximum(c1, 1.0)

    feat = jnp.dot(jnp.concatenate([xs0, xs1], axis=1), lfw_ref[...]) + lfb_ref[...]
    nrm = jnp.sqrt(jnp.sum(feat * feat, axis=1, keepdims=True))
    out_ref[...] = feat / (nrm + 1e-10)


_DENSE_CALL = pl.pallas_call(
    _dense_body,
    out_shape=jax.ShapeDtypeStruct((_B, _ES), jnp.float32),
)


def kernel(nodes_flat, adj_flat, batch, lengths, emb, ge_W1, ge_b1, ge_W2,
           ge_b2, ge_Wl, ge_bl, p_Wr, p_Wl, p_b, gf_W1, gf_b1, gf_W2, gf_b2,
           gf_Wl, gf_bl, lf_W, lf_b):
    del lengths
    p = _PROJ_CALL(emb.T, ge_W1)
    h1 = _sc_gather(p, nodes_flat.astype(jnp.int32))
    return _DENSE_CALL(
        adj_flat, h1,
        batch.astype(jnp.int32).reshape(1, _N),
        ge_b1.reshape(1, _ES), ge_W2, ge_b2.reshape(1, _ES),
        ge_Wl, ge_bl.reshape(1, _ES),
        p_Wr, p_Wl, p_b.reshape(1, 1),
        gf_W1, gf_b1.reshape(1, _ES), gf_W2, gf_b2.reshape(1, _ES),
        gf_Wl, gf_bl.reshape(1, _ES),
        lf_W, lf_b.reshape(1, _ES))


# speedup vs baseline: 1.1564x; 1.1564x over previous
import jax
import jax.numpy as jnp
from jax import lax
from jax.experimental import pallas as pl
from jax.experimental.pallas import tpu as pltpu
from jax.experimental.pallas import tpu_sc as plsc

_N = 512
_B = 16
_ES = 128
_WD = 300
_V = 30000
_RATIO = 0.2
_HI = lax.Precision.HIGHEST
_VCHUNK = 7680

def _proj_body(embt_ref, w1_ref, out_ref):
    out_ref[...] = lax.dot_general(
        embt_ref[...], w1_ref[...],
        dimension_numbers=(((0,), (0,)), ((), ())))


_PROJ_CALL = pl.pallas_call(
    _proj_body,
    grid=(_V // _VCHUNK + (_V % _VCHUNK != 0),),
    in_specs=[
        pl.BlockSpec((_WD, _VCHUNK), lambda j: (0, j)),
        pl.BlockSpec((_WD, _ES), lambda j: (0, 0)),
    ],
    out_specs=pl.BlockSpec((_VCHUNK, _ES), lambda j: (j, 0)),
    out_shape=jax.ShapeDtypeStruct((_V, _ES), jnp.float32),
)

_NW = 32
_RW = _N // _NW


def _sc_gather_body(table, idx_hbm, out, idx_v, rows_v, sem):
    wid = lax.axis_index("s") * 2 + lax.axis_index("c")
    base = wid * _RW
    pltpu.sync_copy(idx_hbm.at[pl.ds(base, _RW)], idx_v)
    pltpu.async_copy(table.at[idx_v], rows_v, sem).wait()
    pltpu.sync_copy(rows_v, out.at[pl.ds(base, _RW)])


def _sc_gather(table, idx):
    k = pl.kernel(
        _sc_gather_body,
        out_type=jax.ShapeDtypeStruct((_N, _ES), jnp.float32),
        mesh=plsc.VectorSubcoreMesh(core_axis_name="c", subcore_axis_name="s"),
        scratch_types=[
            pltpu.VMEM((_RW,), jnp.int32),
            pltpu.VMEM((_RW, _ES), jnp.float32),
            pltpu.SemaphoreType.DMA,
        ],
    )
    return k(table, idx)


def _dense_body(adj_ref, h1_ref, brow_ref,
                b1_ref, w2_ref, b2_ref, wl_ref, bl_ref,
                pwr_ref, pwl_ref, pb_ref,
                f1_ref, fb1_ref, f2_ref, fb2_ref, fl_ref, fbl_ref,
                lfw_ref, lfb_ref, out_ref):
    f32 = jnp.float32
    adj = adj_ref[...]
    w = ((adj + adj.T) > 0).astype(f32)
    wb = w.astype(jnp.bfloat16)
    deg = jnp.sum(w, axis=1, keepdims=True) + 1.0
    dinv = lax.rsqrt(deg)
    dinv2 = dinv * dinv

    def wdot(v):
        v1 = v.astype(jnp.bfloat16)
        r1 = v - v1.astype(f32)
        v2 = r1.astype(jnp.bfloat16)
        v3 = (r1 - v2.astype(f32)).astype(jnp.bfloat16)
        d = lambda a: lax.dot_general(wb, a, (((1,), (0,)), ((), ())),
                                      preferred_element_type=f32)
        return d(v1) + d(v2) + d(v3)

    def gcn_h(h, b, dv, dv2, keep):
        v = h * dv
        if keep is not None:
            v = v * keep
        u = wdot(v)
        if keep is not None:
            u = u * keep
        return u * dv + h * dv2 + b

    relu = lambda t: jnp.maximum(t, 0.0)

    x1 = relu(gcn_h(h1_ref[...], b1_ref[...], dinv, dinv2, None))
    h2 = jnp.dot(x1, w2_ref[...])
    x2 = relu(gcn_h(h2, b2_ref[...], dinv, dinv2, None))
    x = relu(jnp.dot(jnp.concatenate([x1, x2], axis=1), wl_ref[...])
             + bl_ref[...])

    brow = brow_ref[...]
    bcol = brow.T
    m_bn = (lax.broadcasted_iota(jnp.int32, (_B, _N), 0) == brow).astype(f32)
    m_nb = (lax.broadcasted_iota(jnp.int32, (_N, _B), 1) == bcol).astype(f32)
    counts = jnp.sum(m_bn, axis=1, keepdims=True)
    xs0 = jnp.dot(m_bn, x, precision=_HI) / jnp.maximum(counts, 1.0)

    aggr = wdot(x)
    score = (jnp.dot(aggr, pwl_ref[...])
             + jnp.dot(x, pwr_ref[...]) + pb_ref[...])
    score_row = score.T

    kk = jnp.ceil(_RATIO * counts)
    k_node = jnp.dot(m_nb, kk, precision=_HI)

    same = bcol == brow
    ii = lax.broadcasted_iota(jnp.int32, (_N, _N), 0)
    jj = lax.broadcasted_iota(jnp.int32, (_N, _N), 1)
    beats = (score_row > score) | ((score_row == score) & (jj < ii))
    rank = jnp.sum(jnp.where(same & beats, 1.0, 0.0), axis=1, keepdims=True)
    keep = (rank < k_node).astype(f32)

    xg = x * jnp.tanh(score)
    deg2 = 1.0 + keep * jnp.sum(w * keep.T, axis=1, keepdims=True)
    db = lax.rsqrt(deg2)
    db2 = db * db

    g1 = jnp.dot(xg, f1_ref[...])
    y1 = relu(gcn_h(g1, fb1_ref[...], db, db2, keep))
    g2 = jnp.dot(y1, f2_ref[...])
    y2 = relu(gcn_h(g2, fb2_ref[...], db, db2, keep))
    out2 = jnp.dot(jnp.concatenate([y1, y2], axis=1), fl_ref[...]) + fbl_ref[...]

    c1 = jnp.dot(m_bn, keep, precision=_HI)
    xs1 = jnp.dot(m_bn, out2 * keep, precision=_HI) / jnp.maximum(c1, 1.0)

    feat = jnp.dot(jnp.concatenate([xs0, xs1], axis=1), lfw_ref[...]) + lfb_ref[...]
    nrm = jnp.sqrt(jnp.sum(feat * feat, axis=1, keepdims=True))
    out_ref[...] = feat / (nrm + 1e-10)


_DENSE_CALL = pl.pallas_call(
    _dense_body,
    out_shape=jax.ShapeDtypeStruct((_B, _ES), jnp.float32),
)


def kernel(nodes_flat, adj_flat, batch, lengths, emb, ge_W1, ge_b1, ge_W2,
           ge_b2, ge_Wl, ge_bl, p_Wr, p_Wl, p_b, gf_W1, gf_b1, gf_W2, gf_b2,
           gf_Wl, gf_bl, lf_W, lf_b):
    del lengths
    p = _PROJ_CALL(emb.T, ge_W1)
    h1 = _sc_gather(p, nodes_flat.astype(jnp.int32))
    return _DENSE_CALL(
        adj_flat, h1,
        batch.astype(jnp.int32).reshape(1, _N),
        ge_b1.reshape(1, _ES), ge_W2, ge_b2.reshape(1, _ES),
        ge_Wl, ge_bl.reshape(1, _ES),
        p_Wr, p_Wl, p_b.reshape(1, 1),
        gf_W1, gf_b1.reshape(1, _ES), gf_W2, gf_b2.reshape(1, _ES),
        gf_Wl, gf_bl.reshape(1, _ES),
        lf_W, lf_b.reshape(1, _ES))
